# baseline (speedup 1.0000x reference)
import functools

import jax
import jax.numpy as jnp
from jax import lax
from jax.experimental import pallas as pl
from jax.experimental.pallas import tpu as pltpu
from jax.experimental.pallas import tpu_sc as plsc

N = 10000
E = 320000
D = 128
NC = 2
NS = 16
NW = NC * NS
EC = E // NW
BE = 16000
NB = E // BE


def _logits_body(i_ref, enc_row_ref, wattn_ref, syn_ref, ei_ref,
                 out_ref, col_ref, row_ref, m_ref, z_ref, iout_ref,
                 q_scr, mv_scr, acc_scr):
    b = pl.program_id(0)

    @pl.when(b == 0)
    def _init():
        enc_row = enc_row_ref[pl.ds(i_ref[0] % 8, 1), :]
        q_scr[...] = lax.dot_general(
            enc_row, wattn_ref[...], (((1,), (1,)), ((), ())),
            preferred_element_type=jnp.float32)
        mv_scr[...] = jnp.full((1, 128), -jnp.inf, jnp.float32)
        acc_scr[...] = jnp.zeros((1, 128), jnp.float32)

    bl = lax.dot_general(
        q_scr[...], syn_ref[...], (((1,), (1,)), ((), ())),
        preferred_element_type=jnp.float32)
    out_ref[...] = bl
    row_ref[...] = ei_ref[0:1, :]
    col_ref[...] = ei_ref[1:2, :]
    bm = bl[:, 0:128]
    for k in range(1, BE // 128):
        bm = jnp.maximum(bm, bl[:, 128 * k:128 * (k + 1)])
    m_old = mv_scr[...]
    m_new = jnp.maximum(m_old, bm)
    es = jnp.exp(bl[:, 0:128] - m_new)
    for k in range(1, BE // 128):
        es = es + jnp.exp(bl[:, 128 * k:128 * (k + 1)] - m_new)
    acc_scr[...] = acc_scr[...] * jnp.exp(m_old - m_new) + es
    mv_scr[...] = m_new

    @pl.when(b == NB - 1)
    def _fin():
        m = jnp.max(m_new)
        z = jnp.sum(acc_scr[...] * jnp.exp(m_new - m))
        m_ref[...] = jnp.broadcast_to(m, (128,))
        z_ref[...] = jnp.broadcast_to(z, (128,))
        iout_ref[...] = jnp.broadcast_to(i_ref[0], (128,))


def _logits_call(i_in, enc, wattn, syn, edge_index):
    return pl.pallas_call(
        _logits_body,
        grid_spec=pltpu.PrefetchScalarGridSpec(
            num_scalar_prefetch=1,
            grid=(NB,),
            in_specs=[
                pl.BlockSpec((8, D), lambda b, i_sp: (i_sp[0] // 8, 0)),
                pl.BlockSpec((D, D), lambda b, i_sp: (0, 0)),
                pl.BlockSpec((BE, D), lambda b, i_sp: (b, 0)),
                pl.BlockSpec((2, BE), lambda b, i_sp: (0, b)),
            ],
            out_specs=[
                pl.BlockSpec((1, BE), lambda b, i_sp: (0, b)),
                pl.BlockSpec((1, BE), lambda b, i_sp: (0, b)),
                pl.BlockSpec((1, BE), lambda b, i_sp: (0, b)),
                pl.BlockSpec((128,), lambda b, i_sp: (0,)),
                pl.BlockSpec((128,), lambda b, i_sp: (0,)),
                pl.BlockSpec((128,), lambda b, i_sp: (0,)),
            ],
            scratch_shapes=[
                pltpu.VMEM((1, D), jnp.float32),
                pltpu.VMEM((1, 128), jnp.float32),
                pltpu.VMEM((1, 128), jnp.float32),
            ],
        ),
        out_shape=[
            jax.ShapeDtypeStruct((1, E), jnp.float32),
            jax.ShapeDtypeStruct((1, E), jnp.int32),
            jax.ShapeDtypeStruct((1, E), jnp.int32),
            jax.ShapeDtypeStruct((128,), jnp.float32),
            jax.ShapeDtypeStruct((128,), jnp.float32),
            jax.ShapeDtypeStruct((128,), jnp.int32),
        ],
    )(i_in, enc, wattn, syn, edge_index)


def _scatter_body(logits_hbm, col_hbm, row_hbm, m_hbm, z_hbm, i_hbm,
                  degp_hbm, gp_hbm,
                  l_v, c_v, r_v, deg_l, g_l, m_v, z_v, i_v, sem):
    wid = lax.axis_index("s") * NC + lax.axis_index("c")
    base = wid * EC
    h1 = pltpu.async_copy(m_hbm.at[pl.ds(0, 16)], m_v, sem)
    h2 = pltpu.async_copy(z_hbm.at[pl.ds(0, 16)], z_v, sem)
    h3 = pltpu.async_copy(i_hbm.at[pl.ds(0, 16)], i_v, sem)
    h4 = pltpu.async_copy(logits_hbm.at[pl.ds(base, EC)], l_v, sem)
    h5 = pltpu.async_copy(col_hbm.at[pl.ds(base, EC)], c_v, sem)
    h6 = pltpu.async_copy(row_hbm.at[pl.ds(base, EC)], r_v, sem)

    @plsc.parallel_loop(0, N // 16, 1, unroll=4)
    def zbody(j):
        sl = pl.ds(j * 16, 16)
        deg_l[sl] = jnp.zeros((16,), jnp.float32)
        g_l[sl] = jnp.zeros((16,), jnp.float32)

    h1.wait()
    h2.wait()
    h3.wait()
    h4.wait()
    h5.wait()
    h6.wait()

    mvec = m_v[...]
    zivec = 1.0 / z_v[...]
    ivec = i_v[...]

    @plsc.parallel_loop(0, EC // 16, 1, unroll=4)
    def body(t):
        sl = pl.ds(t * 16, 16)
        ew = jnp.exp(l_v[sl] - mvec) * zivec
        c = c_v[sl]
        r = r_v[sl]
        plsc.addupdate_scatter(deg_l, [c], ew)
        plsc.addupdate_scatter(g_l, [r], ew, mask=c == ivec)

    ho1 = pltpu.async_copy(deg_l, degp_hbm.at[wid], sem)
    ho2 = pltpu.async_copy(g_l, gp_hbm.at[wid], sem)
    ho1.wait()
    ho2.wait()


def _scatter_call(logits, col, row, m_b, z_b, i_b):
    kfn = functools.partial(
        pl.kernel,
        out_type=[
            jax.ShapeDtypeStruct((NW, N), jnp.float32),
            jax.ShapeDtypeStruct((NW, N), jnp.float32),
        ],
        mesh=plsc.VectorSubcoreMesh(core_axis_name="c", subcore_axis_name="s"),
        compiler_params=pltpu.CompilerParams(needs_layout_passes=False),
        scratch_types=[
            pltpu.VMEM((EC,), jnp.float32),
            pltpu.VMEM((EC,), jnp.int32),
            pltpu.VMEM((EC,), jnp.int32),
            pltpu.VMEM((N,), jnp.float32),
            pltpu.VMEM((N,), jnp.float32),
            pltpu.VMEM((16,), jnp.float32),
            pltpu.VMEM((16,), jnp.float32),
            pltpu.VMEM((16,), jnp.int32),
            pltpu.SemaphoreType.DMA,
        ],
    )(_scatter_body)
    return kfn(logits, col, row, m_b, z_b, i_b)


def _final_body(i_ref, bo_ref, degp, gp, enc, wg, bg, cau, eff, wo,
                out_ref):
    deg = 1.0 + jnp.sum(degp[...], axis=0, keepdims=True)
    dis = lax.rsqrt(deg)
    a = jnp.sum(gp[...], axis=0, keepdims=True) * dis
    v = lax.dot_general(a, enc[...], (((1,), (0,)), ((), ())),
                        preferred_element_type=jnp.float32)
    lane = lax.broadcasted_iota(jnp.int32, (1, N), 1)
    onehot = (lane == i_ref[0]).astype(jnp.float32)
    dis_i = jnp.sum(onehot * dis)
    enc_i = enc[pl.ds(i_ref[0], 1), :]
    u = dis_i * v + (dis_i * dis_i) * enc_i
    outv = lax.dot_general(u, wg[...], (((1,), (1,)), ((), ())),
                           preferred_element_type=jnp.float32) + bg[...]
    w = wo[...]
    sacc = (jnp.sum(outv * w[:, 0:D]) + jnp.sum(cau[...] * w[:, D:2 * D])
            + jnp.sum(eff[...] * w[:, 2 * D:3 * D]) + bo_ref[0])
    out_ref[...] = jnp.broadcast_to(jax.nn.sigmoid(sacc), (1, 1))


def _final_call(i_in, b_out, degp, gp, enc, wg, bg, cau, eff, wo):
    return pl.pallas_call(
        _final_body,
        in_specs=[
            pl.BlockSpec(memory_space=pltpu.SMEM),
            pl.BlockSpec(memory_space=pltpu.SMEM),
            pl.BlockSpec((NW, N), lambda: (0, 0)),
            pl.BlockSpec((NW, N), lambda: (0, 0)),
            pl.BlockSpec((N, D), lambda: (0, 0)),
            pl.BlockSpec((D, D), lambda: (0, 0)),
            pl.BlockSpec((1, D), lambda: (0, 0)),
            pl.BlockSpec((1, D), lambda: (0, 0)),
            pl.BlockSpec((1, D), lambda: (0, 0)),
            pl.BlockSpec((1, 3 * D), lambda: (0, 0)),
        ],
        out_shape=jax.ShapeDtypeStruct((1, 1), jnp.float32),
    )(i_in, b_out, degp, gp, enc, wg, bg, cau, eff, wo)


def kernel(i, encoder_outputs, syn_embeddeds, cause, effect, edge_index,
           W_attn, W_gcn, b_gcn, W_out, b_out):
    i_in = jnp.asarray(i, jnp.int32).reshape(1)
    logits2, col2, row2, m_b, z_b, i_b = _logits_call(
        i_in, encoder_outputs, W_attn, syn_embeddeds, edge_index)
    degp, gp = _scatter_call(logits2.reshape(E), col2.reshape(E),
                             row2.reshape(E), m_b, z_b, i_b)
    res = _final_call(
        i_in, b_out, degp, gp, encoder_outputs, W_gcn,
        b_gcn.reshape(1, D), cause.reshape(1, D), effect.reshape(1, D),
        W_out)
    return res.reshape(1)


# speedup vs baseline: 1.2008x; 1.2008x over previous
import functools

import jax
import jax.numpy as jnp
from jax import lax
from jax.experimental import pallas as pl
from jax.experimental.pallas import tpu as pltpu
from jax.experimental.pallas import tpu_sc as plsc

N = 10000
E = 320000
D = 128
NC = 2
NS = 16
NW = NC * NS
EC = E // NW
BE = 16000
NB = E // BE


def _logits_body(i_ref, enc_row_ref, wattn_ref, syn_ref,
                 out_ref, m_ref, z_ref, iout_ref, q_scr, mv_scr, acc_scr):
    b = pl.program_id(0)

    @pl.when(b == 0)
    def _init():
        enc_row = enc_row_ref[pl.ds(i_ref[0] % 8, 1), :]
        q_scr[...] = lax.dot_general(
            enc_row, wattn_ref[...], (((1,), (1,)), ((), ())),
            preferred_element_type=jnp.float32)
        mv_scr[...] = jnp.full((1, 128), -jnp.inf, jnp.float32)
        acc_scr[...] = jnp.zeros((1, 128), jnp.float32)

    bl = lax.dot_general(
        q_scr[...], syn_ref[...], (((1,), (1,)), ((), ())),
        preferred_element_type=jnp.float32)
    out_ref[...] = bl[:, None, :]
    bm = bl[:, 0:128]
    for k in range(1, BE // 128):
        bm = jnp.maximum(bm, bl[:, 128 * k:128 * (k + 1)])
    m_old = mv_scr[...]
    m_new = jnp.maximum(m_old, bm)
    es = jnp.exp(bl[:, 0:128] - m_new)
    for k in range(1, BE // 128):
        es = es + jnp.exp(bl[:, 128 * k:128 * (k + 1)] - m_new)
    acc_scr[...] = acc_scr[...] * jnp.exp(m_old - m_new) + es
    mv_scr[...] = m_new

    @pl.when(b == NB - 1)
    def _fin():
        m = jnp.max(m_new)
        z = jnp.sum(acc_scr[...] * jnp.exp(m_new - m))
        m_ref[...] = jnp.broadcast_to(m, (1, 128))
        z_ref[...] = jnp.broadcast_to(z, (1, 128))
        iout_ref[...] = jnp.broadcast_to(i_ref[0], (1, 128))


def _logits_call(i_in, enc, wattn, syn):
    return pl.pallas_call(
        _logits_body,
        grid_spec=pltpu.PrefetchScalarGridSpec(
            num_scalar_prefetch=1,
            grid=(NB,),
            in_specs=[
                pl.BlockSpec((8, D), lambda b, i_sp: (i_sp[0] // 8, 0)),
                pl.BlockSpec((D, D), lambda b, i_sp: (0, 0)),
                pl.BlockSpec((BE, D), lambda b, i_sp: (b, 0)),
            ],
            out_specs=[
                pl.BlockSpec((1, 1, BE), lambda b, i_sp: (b, 0, 0)),
                pl.BlockSpec((1, 128), lambda b, i_sp: (0, 0)),
                pl.BlockSpec((1, 128), lambda b, i_sp: (0, 0)),
                pl.BlockSpec((1, 128), lambda b, i_sp: (0, 0)),
            ],
            scratch_shapes=[
                pltpu.VMEM((1, D), jnp.float32),
                pltpu.VMEM((1, 128), jnp.float32),
                pltpu.VMEM((1, 128), jnp.float32),
            ],
        ),
        out_shape=[
            jax.ShapeDtypeStruct((NB, 1, BE), jnp.float32),
            jax.ShapeDtypeStruct((1, 128), jnp.float32),
            jax.ShapeDtypeStruct((1, 128), jnp.float32),
            jax.ShapeDtypeStruct((1, 128), jnp.int32),
        ],
    )(i_in, enc, wattn, syn)


def _scatter_body(logits_hbm, col_hbm, row_hbm, m_hbm, z_hbm, i_hbm,
                  degp_hbm, gp_hbm,
                  l_v, c_v, r_v, deg_l, g_l, m_v, z_v, i_v, sem):
    wid = lax.axis_index("s") * NC + lax.axis_index("c")
    base = wid * EC
    h1 = pltpu.async_copy(m_hbm.at[0, pl.ds(0, 16)], m_v, sem)
    h2 = pltpu.async_copy(z_hbm.at[0, pl.ds(0, 16)], z_v, sem)
    h3 = pltpu.async_copy(i_hbm.at[0, pl.ds(0, 16)], i_v, sem)
    h4 = pltpu.async_copy(logits_hbm.at[pl.ds(base, EC)], l_v, sem)
    h5 = pltpu.async_copy(col_hbm.at[pl.ds(base, EC)], c_v, sem)
    h6 = pltpu.async_copy(row_hbm.at[pl.ds(base, EC)], r_v, sem)

    @plsc.parallel_loop(0, N // 16, 1, unroll=4)
    def zbody(j):
        sl = pl.ds(j * 16, 16)
        deg_l[sl] = jnp.zeros((16,), jnp.float32)
        g_l[sl] = jnp.zeros((16,), jnp.float32)

    h1.wait()
    h2.wait()
    h3.wait()
    h4.wait()
    h5.wait()
    h6.wait()

    mvec = m_v[...]
    zivec = 1.0 / z_v[...]
    ivec = i_v[...]

    @plsc.parallel_loop(0, EC // 16, 1, unroll=4)
    def body(t):
        sl = pl.ds(t * 16, 16)
        ew = jnp.exp(l_v[sl] - mvec) * zivec
        c = c_v[sl]
        r = r_v[sl]
        plsc.addupdate_scatter(deg_l, [c], ew)
        plsc.addupdate_scatter(g_l, [r], ew, mask=c == ivec)

    ho1 = pltpu.async_copy(deg_l, degp_hbm.at[wid], sem)
    ho2 = pltpu.async_copy(g_l, gp_hbm.at[wid], sem)
    ho1.wait()
    ho2.wait()


def _scatter_call(logits, col, row, m_b, z_b, i_b):
    kfn = functools.partial(
        pl.kernel,
        out_type=[
            jax.ShapeDtypeStruct((NW, N), jnp.float32),
            jax.ShapeDtypeStruct((NW, N), jnp.float32),
        ],
        mesh=plsc.VectorSubcoreMesh(core_axis_name="c", subcore_axis_name="s"),
        compiler_params=pltpu.CompilerParams(needs_layout_passes=False),
        scratch_types=[
            pltpu.VMEM((EC,), jnp.float32),
            pltpu.VMEM((EC,), jnp.int32),
            pltpu.VMEM((EC,), jnp.int32),
            pltpu.VMEM((N,), jnp.float32),
            pltpu.VMEM((N,), jnp.float32),
            pltpu.VMEM((16,), jnp.float32),
            pltpu.VMEM((16,), jnp.float32),
            pltpu.VMEM((16,), jnp.int32),
            pltpu.SemaphoreType.DMA,
        ],
    )(_scatter_body)
    return kfn(logits, col, row, m_b, z_b, i_b)


def _final_body(i_ref, bo_ref, degp, gp, enc, wg, bg, cau, eff, wo,
                out_ref):
    deg = 1.0 + jnp.sum(degp[...], axis=0, keepdims=True)
    dis = lax.rsqrt(deg)
    a = jnp.sum(gp[...], axis=0, keepdims=True) * dis
    v = lax.dot_general(a, enc[...], (((1,), (0,)), ((), ())),
                        preferred_element_type=jnp.float32)
    lane = lax.broadcasted_iota(jnp.int32, (1, N), 1)
    onehot = (lane == i_ref[0]).astype(jnp.float32)
    dis_i = jnp.sum(onehot * dis)
    enc_i = enc[pl.ds(i_ref[0], 1), :]
    u = dis_i * v + (dis_i * dis_i) * enc_i
    outv = lax.dot_general(u, wg[...], (((1,), (1,)), ((), ())),
                           preferred_element_type=jnp.float32) + bg[...]
    w = wo[...]
    sacc = (jnp.sum(outv * w[:, 0:D]) + jnp.sum(cau[...] * w[:, D:2 * D])
            + jnp.sum(eff[...] * w[:, 2 * D:3 * D]) + bo_ref[0])
    out_ref[...] = jnp.broadcast_to(jax.nn.sigmoid(sacc), (1, 1))


def _final_call(i_in, b_out, degp, gp, enc, wg, bg, cau, eff, wo):
    return pl.pallas_call(
        _final_body,
        in_specs=[
            pl.BlockSpec(memory_space=pltpu.SMEM),
            pl.BlockSpec(memory_space=pltpu.SMEM),
            pl.BlockSpec((NW, N), lambda: (0, 0)),
            pl.BlockSpec((NW, N), lambda: (0, 0)),
            pl.BlockSpec((N, D), lambda: (0, 0)),
            pl.BlockSpec((D, D), lambda: (0, 0)),
            pl.BlockSpec((1, D), lambda: (0, 0)),
            pl.BlockSpec((1, D), lambda: (0, 0)),
            pl.BlockSpec((1, D), lambda: (0, 0)),
            pl.BlockSpec((1, 3 * D), lambda: (0, 0)),
        ],
        out_shape=jax.ShapeDtypeStruct((1, 1), jnp.float32),
    )(i_in, b_out, degp, gp, enc, wg, bg, cau, eff, wo)


def kernel(i, encoder_outputs, syn_embeddeds, cause, effect, edge_index,
           W_attn, W_gcn, b_gcn, W_out, b_out):
    i_in = jnp.asarray(i, jnp.int32).reshape(1)
    logits3, m_b, z_b, i_b = _logits_call(i_in, encoder_outputs, W_attn,
                                          syn_embeddeds)
    row = edge_index[0]
    col = edge_index[1]
    degp, gp = _scatter_call(logits3.reshape(E), col, row, m_b, z_b, i_b)
    res = _final_call(
        i_in, b_out, degp, gp, encoder_outputs, W_gcn,
        b_gcn.reshape(1, D), cause.reshape(1, D), effect.reshape(1, D),
        W_out)
    return res.reshape(1)
